# baseline (speedup 1.0000x reference)
import functools

import jax
import jax.numpy as jnp
from jax import lax
from jax.experimental import pallas as pl
from jax.experimental.pallas import tpu as pltpu
from jax.experimental.pallas import tpu_sc as plsc

_K = 1024
_M = 32
_V = 50000
_NC = 2
_NS = 16
_NW = _NC * _NS
_NTR = _V // 8
_TRB = _NTR // _NW
_NBIG = _NTR - _TRB * _NW
_CTR = 5
_NCH = _TRB // _CTR
_CIDX = _CTR * 8 * _M
_NDMA = _CIDX // 128
_BT = 1024


def _argmax_body(wf_ref, wb_ref, idx_ref):
    iota = lax.broadcasted_iota(jnp.int32, (_M, _K), 1)
    for m, ref in enumerate((wf_ref, wb_ref)):
        w = ref[...]
        mx = jnp.max(w, axis=1, keepdims=True)
        first = jnp.min(jnp.where(w == mx, iota, _K), axis=1)
        idx_ref[m, :] = first


def _landmark_indices(W_fwd, W_bwd):
    return pl.pallas_call(
        _argmax_body,
        out_shape=jax.ShapeDtypeStruct((2, _M), jnp.int32),
    )(W_fwd, W_bwd)


def _sc_body(d_in_hbm, idx_hbm, yb_hbm, idxv, idx0, idx1, db0, db1,
             sem0, sem1):
    w = lax.axis_index("s") * _NC + lax.axis_index("c")
    pltpu.sync_copy(idx_hbm, idxv)

    def coff(vec):
        return (lax.shift_left(lax.shift_right_logical(vec, 7), 10)
                + lax.bitwise_and(vec, 127))
    cb = [coff(idxv[1, pl.ds(0, 16)]), coff(idxv[1, pl.ds(16, 16)])]

    trb = w * _TRB + jnp.minimum(w, _NBIG)
    par = ((idx0, db0, sem0), (idx1, db1, sem1))

    def build_rows(tr_abs, ntr, idxr):
        for tr8 in range(ntr):
            base = (tr_abs + tr8) * 8192
            for sr in range(8):
                p = (tr8 * 8 + sr) * _M
                rbv = jnp.full((16,), base + sr * 128, jnp.int32)
                idxr[p // 128, pl.ds(p % 128, 16)] = rbv + cb[0]
                idxr[p // 128, pl.ds(p % 128 + 16, 16)] = rbv + cb[1]

    def fire(ch, p):
        idxr, db, sem = par[p]
        build_rows(trb + ch * _CTR, _CTR, idxr)
        for j in range(_NDMA):
            pltpu.async_copy(
                d_in_hbm.at[idxr.at[j]], db.at[pl.ds(128 * j, 128)], sem)

    def drain_out(ch, p):
        _, db, sem = par[p]
        o0 = (trb + ch * _CTR) * 8 * _M
        pltpu.make_async_copy(d_in_hbm.at[pl.ds(0, _CIDX)], db, sem).wait()
        pltpu.sync_copy(db, yb_hbm.at[pl.ds(o0, _CIDX)])

    fire(0, 0)

    def pair_body(t, carry):
        fire(2 * t + 1, 1)
        drain_out(2 * t, 0)
        fire(2 * t + 2, 0)
        drain_out(2 * t + 1, 1)
        return carry

    lax.fori_loop(0, (_NCH - 1) // 2, pair_body, 0)
    drain_out(_NCH - 1, 0)

    @pl.when(w < _NBIG)
    def _():
        tr0 = trb + _TRB
        build_rows(tr0, 1, idx1)
        copies = [
            pltpu.async_copy(d_in_hbm.at[idx1.at[0]],
                             db1.at[pl.ds(0, 128)], sem1),
            pltpu.async_copy(d_in_hbm.at[idx1.at[1]],
                             db1.at[pl.ds(128, 128)], sem1),
        ]
        for cp in copies:
            cp.wait()
        pltpu.sync_copy(db1.at[pl.ds(0, 256)],
                        yb_hbm.at[pl.ds(tr0 * 8 * _M, 256)])


_sc_gather = functools.partial(
    pl.kernel,
    out_type=jax.ShapeDtypeStruct((_V * _M,), jnp.float32),
    mesh=plsc.VectorSubcoreMesh(
        core_axis_name="c", subcore_axis_name="s",
        num_cores=_NC, num_subcores=_NS),
    compiler_params=pltpu.CompilerParams(use_tc_tiling_on_sc=False),
    scratch_types=[
        pltpu.VMEM((2, _M), jnp.int32),
        pltpu.VMEM((_NDMA, 128), jnp.int32),
        pltpu.VMEM((_NDMA, 128), jnp.int32),
        pltpu.VMEM((_CIDX,), jnp.float32),
        pltpu.VMEM((_CIDX,), jnp.float32),
        pltpu.SemaphoreType.DMA,
        pltpu.SemaphoreType.DMA,
    ],
)(_sc_body)


def _flat_view(d):
    return d.reshape(_V // 8, 8, _K // 128, 128).transpose(
        0, 2, 1, 3).reshape(-1)


def _onehot_from_w(w):
    m, k = w.shape
    iota = lax.broadcasted_iota(jnp.int32, (m, k), 1)
    mx = jnp.max(w, axis=1, keepdims=True)
    first = jnp.min(jnp.where(w == mx, iota, k), axis=1, keepdims=True)
    return (iota == first).astype(jnp.float32)


def _tc_body(d_out_ref, wf_ref, yf_ref):
    pf = _onehot_from_w(wf_ref[...])
    yf_ref[...] = lax.dot_general(
        d_out_ref[...], pf, (((1,), (1,)), ((), ())),
        preferred_element_type=jnp.float32)


def _tc_fwd(d_out_t, W_fwd):
    grid = (pl.cdiv(_V, _BT),)
    return pl.pallas_call(
        _tc_body,
        grid=grid,
        in_specs=[pl.BlockSpec((_BT, _K), lambda i: (i, 0)),
                  pl.BlockSpec((_M, _K), lambda i: (0, 0))],
        out_specs=[pl.BlockSpec((_BT, _M), lambda i: (i, 0))],
        out_shape=[jax.ShapeDtypeStruct((_V, _M), jnp.float32)],
    )(d_out_t, W_fwd)


@jax.jit
def kernel(d_out_t, d_in_t, W_fwd, W_bwd):
    idx_all = _landmark_indices(W_fwd, W_bwd)
    yb_sc = _sc_gather(_flat_view(d_in_t), idx_all)
    (yf,) = _tc_fwd(d_out_t, W_fwd)
    return (yf, yb_sc.reshape(_V, _M))


# speedup vs baseline: 1.0000x; 1.0000x over previous
import functools

import jax
import jax.numpy as jnp
from jax import lax
from jax.experimental import pallas as pl
from jax.experimental.pallas import tpu as pltpu
from jax.experimental.pallas import tpu_sc as plsc

_K = 1024
_M = 32
_V = 50000
_NC = 2
_NS = 16
_NW = _NC * _NS
_NTR = _V // 8
_TRB = _NTR // _NW
_NBIG = _NTR - _TRB * _NW
_CTR = 5
_NCH = _TRB // _CTR
_CIDX = _CTR * 8 * _M
_NDMA = _CIDX // 128
_BT = 1024


def _argmax_body(wf_ref, wb_ref, idx_ref):
    iota = lax.broadcasted_iota(jnp.int32, (_M, _K), 1)
    for m, ref in enumerate((wf_ref, wb_ref)):
        w = ref[...]
        mx = jnp.max(w, axis=1, keepdims=True)
        first = jnp.min(jnp.where(w == mx, iota, _K), axis=1)
        idx_ref[m, :] = first


def _landmark_indices(W_fwd, W_bwd):
    return pl.pallas_call(
        _argmax_body,
        out_shape=jax.ShapeDtypeStruct((2, _M), jnp.int32),
    )(W_fwd, W_bwd)


def _sc_body(d_in_hbm, idx_hbm, yb_hbm, idxv, idx0, idx1, db0, db1,
             sem0, sem1):
    w = lax.axis_index("s") * _NC + lax.axis_index("c")
    pltpu.sync_copy(idx_hbm, idxv)

    def coff(vec):
        return (lax.shift_left(lax.shift_right_logical(vec, 7), 10)
                + lax.bitwise_and(vec, 127))
    cb = [coff(idxv[1, pl.ds(0, 16)]), coff(idxv[1, pl.ds(16, 16)])]

    trb = w * _TRB + jnp.minimum(w, _NBIG)
    par = ((idx0, db0, sem0), (idx1, db1, sem1))

    def build_rows(tr_abs, ntr, idxr):
        for tr8 in range(ntr):
            base = (tr_abs + tr8) * 8192
            for sr in range(8):
                p = (tr8 * 8 + sr) * _M
                rbv = jnp.full((16,), base + sr * 128, jnp.int32)
                idxr[p // 128, pl.ds(p % 128, 16)] = rbv + cb[0]
                idxr[p // 128, pl.ds(p % 128 + 16, 16)] = rbv + cb[1]

    def fire(ch, p):
        idxr, db, sem = par[p]
        build_rows(trb + ch * _CTR, _CTR, idxr)
        for j in range(_NDMA):
            pltpu.async_copy(
                d_in_hbm.at[idxr.at[j]], db.at[pl.ds(128 * j, 128)], sem)

    def drain_out(ch, p):
        _, db, sem = par[p]
        o0 = (trb + ch * _CTR) * 8 * _M
        pltpu.make_async_copy(d_in_hbm.at[pl.ds(0, _CIDX)], db, sem).wait()
        pltpu.sync_copy(db, yb_hbm.at[pl.ds(o0, _CIDX)])

    fire(0, 0)

    def pair_body(t, carry):
        fire(2 * t + 1, 1)
        drain_out(2 * t, 0)
        fire(2 * t + 2, 0)
        drain_out(2 * t + 1, 1)
        return carry

    lax.fori_loop(0, (_NCH - 1) // 2, pair_body, 0)
    drain_out(_NCH - 1, 0)

    @pl.when(w < _NBIG)
    def _():
        tr0 = trb + _TRB
        build_rows(tr0, 1, idx1)
        copies = [
            pltpu.async_copy(d_in_hbm.at[idx1.at[0]],
                             db1.at[pl.ds(0, 128)], sem1),
            pltpu.async_copy(d_in_hbm.at[idx1.at[1]],
                             db1.at[pl.ds(128, 128)], sem1),
        ]
        for cp in copies:
            cp.wait()
        pltpu.sync_copy(db1.at[pl.ds(0, 256)],
                        yb_hbm.at[pl.ds(tr0 * 8 * _M, 256)])


_sc_gather = functools.partial(
    pl.kernel,
    out_type=jax.ShapeDtypeStruct((_V * _M,), jnp.float32),
    mesh=plsc.VectorSubcoreMesh(
        core_axis_name="c", subcore_axis_name="s",
        num_cores=_NC, num_subcores=_NS),
    compiler_params=pltpu.CompilerParams(use_tc_tiling_on_sc=False),
    scratch_types=[
        pltpu.VMEM((2, _M), jnp.int32),
        pltpu.VMEM((_NDMA, 128), jnp.int32),
        pltpu.VMEM((_NDMA, 128), jnp.int32),
        pltpu.VMEM((_CIDX,), jnp.float32),
        pltpu.VMEM((_CIDX,), jnp.float32),
        pltpu.SemaphoreType.DMA,
        pltpu.SemaphoreType.DMA,
    ],
)(_sc_body)


def _flat_view(d):
    return d.reshape(_V // 8, 8, _K // 128, 128).transpose(
        0, 2, 1, 3).reshape(-1)


def _onehot_from_w(w):
    m, k = w.shape
    iota = lax.broadcasted_iota(jnp.int32, (m, k), 1)
    mx = jnp.max(w, axis=1, keepdims=True)
    first = jnp.min(jnp.where(w == mx, iota, k), axis=1, keepdims=True)
    return (iota == first).astype(jnp.float32)


def _tc_body(d_out_ref, wf_ref, yf_ref):
    pf = _onehot_from_w(wf_ref[...])
    yf_ref[...] = lax.dot_general(
        d_out_ref[...], pf, (((1,), (1,)), ((), ())),
        preferred_element_type=jnp.float32)


def _tc_fwd(d_out_t, W_fwd):
    grid = (pl.cdiv(_V, _BT),)
    return pl.pallas_call(
        _tc_body,
        grid=grid,
        in_specs=[pl.BlockSpec((_BT, _K), lambda i: (i, 0)),
                  pl.BlockSpec((_M, _K), lambda i: (0, 0))],
        out_specs=[pl.BlockSpec((_BT, _M), lambda i: (i, 0))],
        out_shape=[jax.ShapeDtypeStruct((_V, _M), jnp.float32)],
    )(d_out_t, W_fwd)


@jax.jit
def kernel(d_out_t, d_in_t, W_fwd, W_bwd):
    idx_all = _landmark_indices(W_fwd, W_bwd)
    (yf,) = _tc_fwd(d_out_t, W_fwd)
    yb_sc = _sc_gather(_flat_view(d_in_t), idx_all)
    return (yf, yb_sc.reshape(_V, _M))


# speedup vs baseline: 1.0160x; 1.0160x over previous
import functools

import jax
import jax.numpy as jnp
from jax import lax
from jax.experimental import pallas as pl
from jax.experimental.pallas import tpu as pltpu
from jax.experimental.pallas import tpu_sc as plsc

_K = 1024
_M = 32
_V = 50000
_NC = 2
_NS = 16
_NW = _NC * _NS
_NTR = _V // 8
_TRB = _NTR // _NW
_NBIG = _NTR - _TRB * _NW
_CTR = 5
_NCH = _TRB // _CTR
_CIDX = _CTR * 8 * _M
_NDMA = _CIDX // 128
_BT = 1024


def _argmax_body(wf_ref, wb_ref, idx_ref):
    iota = lax.broadcasted_iota(jnp.int32, (_M, _K), 1)
    for m, ref in enumerate((wf_ref, wb_ref)):
        w = ref[...]
        mx = jnp.max(w, axis=1, keepdims=True)
        first = jnp.min(jnp.where(w == mx, iota, _K), axis=1)
        idx_ref[m, :] = first


def _landmark_indices(W_fwd, W_bwd):
    return pl.pallas_call(
        _argmax_body,
        out_shape=jax.ShapeDtypeStruct((2, _M), jnp.int32),
    )(W_fwd, W_bwd)


def _sc_body(d_in_hbm, idx_hbm, yb_hbm, idxv, idx0, idx1, db0, db1,
             sem0, sem1):
    w = lax.axis_index("s") * _NC + lax.axis_index("c")
    pltpu.sync_copy(idx_hbm, idxv)

    def coff(vec):
        return (lax.shift_left(lax.shift_right_logical(vec, 7), 10)
                + lax.bitwise_and(vec, 127))
    cb = [coff(idxv[1, pl.ds(0, 16)]), coff(idxv[1, pl.ds(16, 16)])]

    trb = w * _TRB + jnp.minimum(w, _NBIG)
    par = ((idx0, db0, sem0), (idx1, db1, sem1))

    def build_rows(tr_abs, ntr, idxr):
        for tr8 in range(ntr):
            base = (tr_abs + tr8) * 8192
            for sr in range(8):
                p = (tr8 * 8 + sr) * _M
                rbv = jnp.full((16,), base + sr * 128, jnp.int32)
                idxr[p // 128, pl.ds(p % 128, 16)] = rbv + cb[0]
                idxr[p // 128, pl.ds(p % 128 + 16, 16)] = rbv + cb[1]

    def fire(ch, p):
        idxr, db, sem = par[p]
        build_rows(trb + ch * _CTR, _CTR, idxr)
        for j in range(_NDMA):
            pltpu.async_copy(
                d_in_hbm.at[idxr.at[j]], db.at[pl.ds(128 * j, 128)], sem)

    def drain_out(ch, p):
        _, db, sem = par[p]
        o0 = (trb + ch * _CTR) * 8 * _M
        pltpu.make_async_copy(d_in_hbm.at[pl.ds(0, _CIDX)], db, sem).wait()
        pltpu.sync_copy(db, yb_hbm.at[pl.ds(o0, _CIDX)])

    fire(0, 0)

    def pair_body(t, carry):
        fire(2 * t + 1, 1)
        drain_out(2 * t, 0)
        fire(2 * t + 2, 0)
        drain_out(2 * t + 1, 1)
        return carry

    lax.fori_loop(0, (_NCH - 1) // 2, pair_body, 0)
    drain_out(_NCH - 1, 0)

    @pl.when(w < _NBIG)
    def _():
        tr0 = trb + _TRB
        build_rows(tr0, 1, idx1)
        copies = [
            pltpu.async_copy(d_in_hbm.at[idx1.at[0]],
                             db1.at[pl.ds(0, 128)], sem1),
            pltpu.async_copy(d_in_hbm.at[idx1.at[1]],
                             db1.at[pl.ds(128, 128)], sem1),
        ]
        for cp in copies:
            cp.wait()
        pltpu.sync_copy(db1.at[pl.ds(0, 256)],
                        yb_hbm.at[pl.ds(tr0 * 8 * _M, 256)])


_sc_gather = functools.partial(
    pl.kernel,
    out_type=jax.ShapeDtypeStruct((_V * _M,), jnp.float32),
    mesh=plsc.VectorSubcoreMesh(
        core_axis_name="c", subcore_axis_name="s",
        num_cores=_NC, num_subcores=_NS),
    compiler_params=pltpu.CompilerParams(use_tc_tiling_on_sc=False),
    scratch_types=[
        pltpu.VMEM((2, _M), jnp.int32),
        pltpu.VMEM((_NDMA, 128), jnp.int32),
        pltpu.VMEM((_NDMA, 128), jnp.int32),
        pltpu.VMEM((_CIDX,), jnp.float32),
        pltpu.VMEM((_CIDX,), jnp.float32),
        pltpu.SemaphoreType.DMA,
        pltpu.SemaphoreType.DMA,
    ],
)(_sc_body)


def _flat_view(d):
    return d.reshape(_V // 8, 8, _K // 128, 128).transpose(
        0, 2, 1, 3).reshape(-1)


def _onehot_from_w(w):
    m, k = w.shape
    iota = lax.broadcasted_iota(jnp.int32, (m, k), 1)
    mx = jnp.max(w, axis=1, keepdims=True)
    first = jnp.min(jnp.where(w == mx, iota, k), axis=1, keepdims=True)
    return (iota == first).astype(jnp.float32)


def _tc_body(d_out_ref, wf_ref, yf_ref):
    pf = _onehot_from_w(wf_ref[...])
    yf_ref[...] = lax.dot_general(
        d_out_ref[...], pf, (((1,), (1,)), ((), ())),
        preferred_element_type=jnp.float32)


def _tc_fwd(d_out_t, W_fwd):
    grid = (pl.cdiv(_V, _BT),)
    return pl.pallas_call(
        _tc_body,
        grid=grid,
        in_specs=[pl.BlockSpec((_BT, _K), lambda i: (i, 0)),
                  pl.BlockSpec((_M, _K), lambda i: (0, 0))],
        out_specs=[pl.BlockSpec((_BT, _M), lambda i: (i, 0))],
        out_shape=[jax.ShapeDtypeStruct((_V, _M), jnp.float32)],
    )(d_out_t, W_fwd)


def _final_body(yf_in, yb_in, yf_ref, yb_ref):
    yb_ref[...] = yb_in[...]


def _final(yf_tc, yb_2d):
    y_spec = pl.BlockSpec((_BT, _M), lambda i: (i, 0))
    any_spec = pl.BlockSpec(memory_space=pl.ANY)
    return pl.pallas_call(
        _final_body,
        grid=(pl.cdiv(_V, _BT),),
        in_specs=[any_spec, y_spec],
        out_specs=[any_spec, y_spec],
        out_shape=[
            jax.ShapeDtypeStruct((_V, _M), jnp.float32),
            jax.ShapeDtypeStruct((_V, _M), jnp.float32),
        ],
        input_output_aliases={0: 0},
    )(yf_tc, yb_2d)


@jax.jit
def kernel(d_out_t, d_in_t, W_fwd, W_bwd):
    idx_all = _landmark_indices(W_fwd, W_bwd)
    yb_sc = _sc_gather(_flat_view(d_in_t), idx_all)
    (yf_tc,) = _tc_fwd(d_out_t, W_fwd)
    yf, yb = _final(yf_tc, yb_sc.reshape(_V, _M))
    return (yf, yb)


# speedup vs baseline: 1.1073x; 1.0898x over previous
import functools

import jax
import jax.numpy as jnp
from jax import lax
from jax.experimental import pallas as pl
from jax.experimental.pallas import tpu as pltpu
from jax.experimental.pallas import tpu_sc as plsc

_K = 1024
_M = 32
_V = 50000
_NC = 2
_NS = 16
_NW = _NC * _NS
_NTR = _V // 8
_TRB = _NTR // _NW
_NBIG = _NTR - _TRB * _NW
_CTR = 5
_NCH = _TRB // _CTR
_CROWS = _CTR * 8
_CIDX = _CROWS * _M
_NDMA = _CIDX // 128


def _argmax_body(wf_ref, wb_ref, idx_ref):
    iota = lax.broadcasted_iota(jnp.int32, (_M, _K), 1)
    for m, ref in enumerate((wf_ref, wb_ref)):
        w = ref[...]
        mx = jnp.max(w, axis=1, keepdims=True)
        first = jnp.min(jnp.where(w == mx, iota, _K), axis=1)
        idx_ref[m, :] = first


def _landmark_indices(W_fwd, W_bwd):
    return pl.pallas_call(
        _argmax_body,
        out_shape=jax.ShapeDtypeStruct((2, _M), jnp.int32),
    )(W_fwd, W_bwd)


def _sc_body(d_out_hbm, d_in_hbm, idx_hbm, yf_hbm, yb_hbm,
             idxv, idxf0, idxb0, idxf1, idxb1, dbf0, dbb0, dbf1, dbb1,
             semf0, semb0, semf1, semb1):
    w = lax.axis_index("s") * _NC + lax.axis_index("c")
    pltpu.sync_copy(idx_hbm, idxv)

    def coff(vec):
        return (lax.shift_left(lax.shift_right_logical(vec, 7), 10)
                + lax.bitwise_and(vec, 127))
    cf = [coff(idxv[0, pl.ds(0, 16)]), coff(idxv[0, pl.ds(16, 16)])]
    cb = [coff(idxv[1, pl.ds(0, 16)]), coff(idxv[1, pl.ds(16, 16)])]

    trb = w * _TRB + jnp.minimum(w, _NBIG)
    par = ((idxf0, idxb0, dbf0, dbb0, semf0, semb0),
           (idxf1, idxb1, dbf1, dbb1, semf1, semb1))

    def build_rows(tr_abs, ntr, idxf, idxb):
        for tr8 in range(ntr):
            base = (tr_abs + tr8) * 8192
            for sr in range(8):
                p = (tr8 * 8 + sr) * _M
                rbv = jnp.full((16,), base + sr * 128, jnp.int32)
                idxf[p // 128, pl.ds(p % 128, 16)] = rbv + cf[0]
                idxf[p // 128, pl.ds(p % 128 + 16, 16)] = rbv + cf[1]
                idxb[p // 128, pl.ds(p % 128, 16)] = rbv + cb[0]
                idxb[p // 128, pl.ds(p % 128 + 16, 16)] = rbv + cb[1]

    def fire(ch, p):
        idxf, idxb, dbf, dbb, semf, semb = par[p]
        build_rows(trb + ch * _CTR, _CTR, idxf, idxb)
        for j in range(_NDMA):
            pltpu.async_copy(
                d_out_hbm.at[idxf.at[j]], dbf.at[pl.ds(128 * j, 128)], semf)
            pltpu.async_copy(
                d_in_hbm.at[idxb.at[j]], dbb.at[pl.ds(128 * j, 128)], semb)

    def drain_out(ch, p):
        _, _, dbf, dbb, semf, semb = par[p]
        tr0 = trb + ch * _CTR
        pltpu.make_async_copy(d_out_hbm.at[pl.ds(0, _CIDX)], dbf, semf).wait()
        pltpu.make_async_copy(d_in_hbm.at[pl.ds(0, _CIDX)], dbb, semb).wait()
        pltpu.sync_copy(dbf, yf_hbm.at[pl.ds(tr0 * 8 * _M, _CIDX)])
        pltpu.sync_copy(dbb, yb_hbm.at[pl.ds(tr0 * 8 * _M, _CIDX)])

    fire(0, 0)

    def pair_body(t, carry):
        fire(2 * t + 1, 1)
        drain_out(2 * t, 0)
        fire(2 * t + 2, 0)
        drain_out(2 * t + 1, 1)
        return carry

    lax.fori_loop(0, (_NCH - 1) // 2, pair_body, 0)
    drain_out(_NCH - 1, 0)

    @pl.when(w < _NBIG)
    def _():
        tr0 = trb + _TRB
        build_rows(tr0, 1, idxf1, idxb1)
        copies = [
            pltpu.async_copy(d_out_hbm.at[idxf1.at[0]],
                             dbf1.at[pl.ds(0, 128)], semf1),
            pltpu.async_copy(d_out_hbm.at[idxf1.at[1]],
                             dbf1.at[pl.ds(128, 128)], semf1),
            pltpu.async_copy(d_in_hbm.at[idxb1.at[0]],
                             dbb1.at[pl.ds(0, 128)], semb1),
            pltpu.async_copy(d_in_hbm.at[idxb1.at[1]],
                             dbb1.at[pl.ds(128, 128)], semb1),
        ]
        for cp in copies:
            cp.wait()
        pltpu.sync_copy(dbf1.at[pl.ds(0, 256)],
                        yf_hbm.at[pl.ds(tr0 * 8 * _M, 256)])
        pltpu.sync_copy(dbb1.at[pl.ds(0, 256)],
                        yb_hbm.at[pl.ds(tr0 * 8 * _M, 256)])


_sc_gather = functools.partial(
    pl.kernel,
    out_type=[
        jax.ShapeDtypeStruct((_V * _M,), jnp.float32),
        jax.ShapeDtypeStruct((_V * _M,), jnp.float32),
    ],
    mesh=plsc.VectorSubcoreMesh(
        core_axis_name="c", subcore_axis_name="s",
        num_cores=_NC, num_subcores=_NS),
    compiler_params=pltpu.CompilerParams(use_tc_tiling_on_sc=False),
    scratch_types=[
        pltpu.VMEM((2, _M), jnp.int32),
        pltpu.VMEM((_NDMA, 128), jnp.int32),
        pltpu.VMEM((_NDMA, 128), jnp.int32),
        pltpu.VMEM((_NDMA, 128), jnp.int32),
        pltpu.VMEM((_NDMA, 128), jnp.int32),
        pltpu.VMEM((_CIDX,), jnp.float32),
        pltpu.VMEM((_CIDX,), jnp.float32),
        pltpu.VMEM((_CIDX,), jnp.float32),
        pltpu.VMEM((_CIDX,), jnp.float32),
        pltpu.SemaphoreType.DMA,
        pltpu.SemaphoreType.DMA,
        pltpu.SemaphoreType.DMA,
        pltpu.SemaphoreType.DMA,
    ],
)(_sc_body)


def _flat_view(d):
    return d.reshape(_V // 8, 8, _K // 128, 128).transpose(
        0, 2, 1, 3).reshape(-1)


@jax.jit
def kernel(d_out_t, d_in_t, W_fwd, W_bwd):
    idx_all = _landmark_indices(W_fwd, W_bwd)
    yf, yb = _sc_gather(_flat_view(d_out_t), _flat_view(d_in_t), idx_all)
    return (yf.reshape(_V, _M), yb.reshape(_V, _M))
